# baseline (speedup 1.0000x reference)
import functools

import jax
import jax.numpy as jnp
from jax import lax
from jax.experimental import pallas as pl
from jax.experimental.pallas import tpu as pltpu
from jax.experimental.pallas import tpu_sc as plsc

B = 16384
L = 200
V = 26
VP = 32
CP = 128
WPS = L // 4
WROWS = 16384 * (L // 4) // 128
NC, NS, LANES = 2, 16, 16
NW = NC * NS
SPW = B // NW
CHUNK = 512
NCHUNK = SPW // CHUNK
GROUPS = CHUNK // LANES


def _sc_hist_kernel(xw_hbm, out_hbm, x_buf0, counts_buf, sem0):
  wid = lax.axis_index("s") * NC + lax.axis_index("c")
  wb = wid * SPW

  iota = lax.broadcasted_iota(jnp.int32, (LANES,), 0)
  ones = jnp.full((LANES,), 1.0, dtype=jnp.float32)
  zeros = jnp.zeros((LANES,), dtype=jnp.float32)
  sems = (sem0,)
  bufs = (x_buf0,)

  crows = CHUNK * WPS // 128

  def start(c):
    return pltpu.async_copy(
        xw_hbm.at[pl.ds(wid * (SPW * WPS // 128) + c * crows, crows), :],
        bufs[c % NCHUNK], sems[c % NCHUNK])

  cp = start(0)

  @plsc.parallel_loop(0, SPW, unroll=4)
  def _(i):
    counts_buf[i, pl.ds(0, LANES)] = zeros
    counts_buf[i, pl.ds(LANES, LANES)] = zeros

  for c in range(NCHUNK):
    nxt = start(c + 1) if c + 1 < NCHUNK else None
    cp.wait()
    buf = bufs[c % NCHUNK]

    def group_body(g, _):
      samp_off = (g * LANES + iota) * WPS
      rows = c * CHUNK + g * LANES + iota

      @plsc.parallel_loop(0, WPS, unroll=4)
      def _(lw):
        wi = samp_off + lw
        wv = plsc.load_gather(buf, [wi >> 7, wi & 127])
        for k in range(4):
          val = (wv >> (8 * k)) & 0xFF
          plsc.addupdate_scatter(counts_buf, [rows, val], ones)
      return 0
    lax.fori_loop(0, GROUPS, group_body, 0)
    cp = nxt

  pltpu.sync_copy(counts_buf, out_hbm.at[pl.ds(wb, SPW), :])


@jax.jit
def _sc_hist(x_words):
  mesh = plsc.VectorSubcoreMesh(core_axis_name="c", subcore_axis_name="s")
  fn = functools.partial(
      pl.kernel,
      mesh=mesh,
      compiler_params=pltpu.CompilerParams(needs_layout_passes=False),
      out_type=jax.ShapeDtypeStruct((B, CP), jnp.float32),
      scratch_types=[
          pltpu.VMEM((CHUNK * WPS // 128, 128), jnp.int32),
          pltpu.VMEM((SPW, CP), jnp.float32),
          pltpu.SemaphoreType.DMA,
      ],
  )(_sc_hist_kernel)
  return fn(x_words)


BLK = 2048
OP = 8


def _tc_mlp_kernel(counts_ref, emb_ref, w1_ref, b1_ref, w2_ref, b2_ref,
                   out_ref):
  cnt = counts_ref[...]
  lane = lax.broadcasted_iota(jnp.int32, cnt.shape, 1)
  cnt = jnp.where(lane < VP, cnt, 0.0)
  pooled = jnp.dot(cnt, emb_ref[...], precision=lax.Precision.HIGHEST,
                   preferred_element_type=jnp.float32)
  h = jnp.dot(pooled, w1_ref[...], preferred_element_type=jnp.float32)
  h = jnp.maximum(h + b1_ref[...], 0.0)
  out_ref[...] = (
      jnp.dot(h, w2_ref[...], preferred_element_type=jnp.float32)
      + b2_ref[...])


@jax.jit
def _tc_mlp(counts2d, emb_pad, W1, b1r, W2p, b2p):
  h = W1.shape[1]
  return pl.pallas_call(
      _tc_mlp_kernel,
      grid=(B // BLK,),
      in_specs=[
          pl.BlockSpec((BLK, CP), lambda i: (i, 0)),
          pl.BlockSpec((CP, emb_pad.shape[1]), lambda i: (0, 0)),
          pl.BlockSpec((emb_pad.shape[1], h), lambda i: (0, 0)),
          pl.BlockSpec((1, h), lambda i: (0, 0)),
          pl.BlockSpec((h, OP), lambda i: (0, 0)),
          pl.BlockSpec((1, OP), lambda i: (0, 0)),
      ],
      out_specs=pl.BlockSpec((BLK, OP), lambda i: (i, 0)),
      out_shape=jax.ShapeDtypeStruct((B, OP), jnp.float32),
  )(counts2d, emb_pad, W1, b1r, W2p, b2p)


def kernel(x, emb, W1, b1, W2, b2):
  x4 = x.astype(jnp.uint32)
  xw = (x4[:, 0::4] | (x4[:, 1::4] << 8) | (x4[:, 2::4] << 16)
        | (x4[:, 3::4] << 24))
  xw = xw.astype(jnp.int32).reshape(WROWS, 128)
  counts = _sc_hist(xw)

  d = emb.shape[1]
  o = W2.shape[1]
  emb_pad = jnp.zeros((CP, d), jnp.float32).at[:V].set(emb)
  W2p = jnp.zeros((W2.shape[0], OP), jnp.float32).at[:, :o].set(W2)
  b2p = jnp.zeros((1, OP), jnp.float32).at[:, :o].set(b2)

  out = _tc_mlp(counts, emb_pad, W1, b1.reshape(1, -1), W2p, b2p)
  return out[:, :o]


# speedup vs baseline: 2.0188x; 2.0188x over previous
import functools

import jax
import jax.numpy as jnp
from jax import lax
from jax.experimental import pallas as pl
from jax.experimental.pallas import tpu as pltpu
from jax.experimental.pallas import tpu_sc as plsc

B = 16384
L = 200
V = 26
VP = 32
CP = 128
WPS = L // 4
WROWS = 16384 * (L // 4) // 128
NC, NS, LANES = 2, 16, 16
NW = NC * NS
SPW = B // NW
CHUNK = 512
NCHUNK = SPW // CHUNK
GROUPS = CHUNK // LANES


def _sc_hist_kernel(xw_hbm, out_hbm, x_buf0, counts_buf, sem0):
  wid = lax.axis_index("s") * NC + lax.axis_index("c")
  wb = wid * SPW

  iota = lax.broadcasted_iota(jnp.int32, (LANES,), 0)
  ones = jnp.full((LANES,), 1.0, dtype=jnp.float32)
  zeros = jnp.zeros((LANES,), dtype=jnp.float32)
  sems = (sem0,)
  bufs = (x_buf0,)

  crows = CHUNK * WPS // 128

  def start(c):
    return pltpu.async_copy(
        xw_hbm.at[pl.ds(wid * (SPW * WPS // 128) + c * crows, crows), :],
        bufs[c % NCHUNK], sems[c % NCHUNK])

  cp = start(0)

  @plsc.parallel_loop(0, SPW, unroll=4)
  def _(i):
    counts_buf[i, pl.ds(0, LANES)] = zeros
    counts_buf[i, pl.ds(LANES, LANES)] = zeros

  for c in range(NCHUNK):
    nxt = start(c + 1) if c + 1 < NCHUNK else None
    cp.wait()
    buf = bufs[c % NCHUNK]

    def group_body(g, _):
      samp_off = (g * LANES + iota) * WPS
      rows = c * CHUNK + g * LANES + iota

      @plsc.parallel_loop(0, WPS, unroll=4)
      def _(lw):
        wi = samp_off + lw
        wv = plsc.load_gather(buf, [wi >> 7, wi & 127])
        for k in range(4):
          val = (wv >> (8 * k)) & 0xFF
          plsc.addupdate_scatter(counts_buf, [rows, val], ones)
      return 0
    lax.fori_loop(0, GROUPS, group_body, 0)
    cp = nxt

  pltpu.sync_copy(counts_buf, out_hbm.at[pl.ds(wb, SPW), :])


@jax.jit
def _sc_hist(x_words):
  mesh = plsc.VectorSubcoreMesh(core_axis_name="c", subcore_axis_name="s")
  fn = functools.partial(
      pl.kernel,
      mesh=mesh,
      compiler_params=pltpu.CompilerParams(needs_layout_passes=False),
      out_type=jax.ShapeDtypeStruct((B, CP), jnp.float32),
      scratch_types=[
          pltpu.VMEM((CHUNK * WPS // 128, 128), jnp.int32),
          pltpu.VMEM((SPW, CP), jnp.float32),
          pltpu.SemaphoreType.DMA,
      ],
  )(_sc_hist_kernel)
  return fn(x_words)


BLK = 2048
OP = 8


def _tc_mlp_kernel(counts_ref, emb_ref, w1_ref, b1_ref, w2_ref, b2_ref,
                   out_ref):
  cnt = counts_ref[...]
  lane = lax.broadcasted_iota(jnp.int32, cnt.shape, 1)
  cnt = jnp.where(lane < VP, cnt, 0.0)
  pooled = jnp.dot(cnt, emb_ref[...], precision=lax.Precision.HIGHEST,
                   preferred_element_type=jnp.float32)
  h = jnp.dot(pooled, w1_ref[...], preferred_element_type=jnp.float32)
  h = jnp.maximum(h + b1_ref[...], 0.0)
  out_ref[...] = (
      jnp.dot(h, w2_ref[...], preferred_element_type=jnp.float32)
      + b2_ref[...])


@jax.jit
def _tc_mlp(counts2d, emb_pad, W1, b1r, W2p, b2p):
  h = W1.shape[1]
  return pl.pallas_call(
      _tc_mlp_kernel,
      grid=(B // BLK,),
      in_specs=[
          pl.BlockSpec((BLK, CP), lambda i: (i, 0)),
          pl.BlockSpec((CP, emb_pad.shape[1]), lambda i: (0, 0)),
          pl.BlockSpec((emb_pad.shape[1], h), lambda i: (0, 0)),
          pl.BlockSpec((1, h), lambda i: (0, 0)),
          pl.BlockSpec((h, OP), lambda i: (0, 0)),
          pl.BlockSpec((1, OP), lambda i: (0, 0)),
      ],
      out_specs=pl.BlockSpec((BLK, OP), lambda i: (i, 0)),
      out_shape=jax.ShapeDtypeStruct((B, OP), jnp.float32),
  )(counts2d, emb_pad, W1, b1r, W2p, b2p)


def kernel(x, emb, W1, b1, W2, b2):
  x4 = x.astype(jnp.uint32)
  q = WPS
  xw = (x4[:, 0:q] | (x4[:, q:2 * q] << 8) | (x4[:, 2 * q:3 * q] << 16)
        | (x4[:, 3 * q:4 * q] << 24))
  xw = xw.astype(jnp.int32).reshape(WROWS, 128)
  counts = _sc_hist(xw)

  d = emb.shape[1]
  o = W2.shape[1]
  emb_pad = jnp.zeros((CP, d), jnp.float32).at[:V].set(emb)
  W2p = jnp.zeros((W2.shape[0], OP), jnp.float32).at[:, :o].set(W2)
  b2p = jnp.zeros((1, OP), jnp.float32).at[:, :o].set(b2)

  out = _tc_mlp(counts, emb_pad, W1, b1.reshape(1, -1), W2p, b2p)
  return out[:, :o]
